# baseline (device time: 41609 ns/iter reference)
import jax
import jax.numpy as jnp
from jax import lax
from jax.experimental import pallas as pl
from jax.experimental.pallas import tpu as pltpu

N_DEV = 16
N_TOK = 2048
D_IN = 512
D_OUT = 1024
N_EXP = 64
E_PER = N_EXP // N_DEV
CAP = 25
CAP_PAD = 32
BLK = E_PER * CAP_PAD
T_PER = N_TOK // N_DEV
H = N_DEV // 2
N_CHUNK = 8
CH = BLK // N_CHUNK
CPE = CAP_PAD // CH

_PERM = (0, 4, 8, 12, 13, 9, 5, 1, 2, 6, 10, 14, 15, 11, 7, 3)
_RINGPOS = [0] * N_DEV
for _p, _d in enumerate(_PERM):
    _RINGPOS[_d] = _p


def _body(x_hbm, ew_hbm, route_ref, route_col_ref, nbr_ref, out_ref,
          x_vmem, ew_vmem, comm_ref, load_sems,
          send_r, recv_r, send_l, recv_l):
    my = lax.axis_index("i")
    right = nbr_ref[0]
    left = nbr_ref[1]

    x_copy = pltpu.make_async_copy(x_hbm, x_vmem, load_sems.at[0])
    x_copy.start()
    ew_copies = []
    for s in range(E_PER):
        cp = pltpu.make_async_copy(ew_hbm.at[s], ew_vmem.at[s],
                                   load_sems.at[1 + s])
        cp.start()
        ew_copies.append(cp)

    barrier_sem = pltpu.get_barrier_semaphore()
    for nbr in (left, right):
        pl.semaphore_signal(barrier_sem, inc=1, device_id=(nbr,),
                            device_id_type=pl.DeviceIdType.MESH)

    route_i = route_ref[...]
    s_iota = lax.broadcasted_iota(jnp.int32, (8, N_TOK), 0)
    m8 = (route_i == my * E_PER + s_iota).astype(jnp.float32)
    incl = m8
    k = 1
    while k < N_TOK:
        incl = incl + jnp.concatenate(
            [jnp.zeros((8, k), jnp.float32), incl[:, :N_TOK - k]], axis=1)
        k *= 2
    excl8 = incl - m8

    pos_exp = jnp.broadcast_to(excl8[0:E_PER, None, :],
                               (E_PER, CAP_PAD, N_TOK)).reshape(BLK, N_TOK)
    m_exp = jnp.broadcast_to(m8[0:E_PER, None, :],
                             (E_PER, CAP_PAD, N_TOK)).reshape(BLK, N_TOK)
    c_of_r = lax.rem(
        lax.broadcasted_iota(jnp.int32, (BLK, N_TOK), 0), CAP_PAD)
    psrc = ((m_exp > 0)
            & (c_of_r.astype(jnp.float32) == pos_exp)
            & (c_of_r < CAP)).astype(jnp.float32)

    pl.semaphore_wait(barrier_sem, 2)

    def mk(src_slot, dst_slot, c, ssem, rsem, dev):
        return pltpu.make_async_remote_copy(
            src_ref=comm_ref.at[src_slot, pl.ds(c * CH, CH)],
            dst_ref=comm_ref.at[dst_slot, pl.ds(c * CH, CH)],
            send_sem=ssem,
            recv_sem=rsem,
            device_id=(dev,),
            device_id_type=pl.DeviceIdType.MESH,
        )

    r_desc = [[None] * N_CHUNK for _ in range(H)]
    l_desc = [[None] * N_CHUNK for _ in range(H - 1)]

    x_copy.wait()
    for s in range(E_PER):
        x_sel_s = jnp.dot(psrc[s * CAP_PAD:(s + 1) * CAP_PAD, :],
                          x_vmem[...],
                          preferred_element_type=jnp.float32)
        ew_copies[s].wait()
        comm_ref[0, s * CAP_PAD:(s + 1) * CAP_PAD, :] = jnp.dot(
            x_sel_s, ew_vmem[s],
            preferred_element_type=jnp.float32).astype(jnp.bfloat16)
        for c in range(s * CPE, (s + 1) * CPE):
            d = mk(0, 1, c, send_r.at[0, c], recv_r.at[0, c], right)
            d.start()
            r_desc[0][c] = d
            d = mk(0, N_DEV - 1, c, send_l.at[0, c], recv_l.at[0, c], left)
            d.start()
            l_desc[0][c] = d

    route_g = route_ref[:, pl.ds(my * T_PER, T_PER)]
    j_iota = lax.broadcasted_iota(jnp.int32, (N_TOK, T_PER), 0)
    i_iota = lax.broadcasted_iota(jnp.int32, (N_TOK, T_PER), 1)
    qt = ((route_col_ref[...] == route_g)
          & (j_iota < my * T_PER + i_iota)).astype(jnp.bfloat16)
    pos_g = jnp.dot(jnp.ones((1, N_TOK), jnp.bfloat16), qt,
                    preferred_element_type=jnp.float32)
    k_g = route_g // E_PER
    w_g = lax.rem(k_g, 4)
    z_g = k_g // 4
    rp_g = w_g * 4 + jnp.where(lax.rem(w_g, 2) == 0, z_g, 3 - z_g)
    my_pos = nbr_ref[2]
    slot_g = lax.rem(my_pos - rp_g + N_DEV, N_DEV)
    gcol_f = (slot_g * BLK
              + lax.rem(route_g, E_PER) * CAP_PAD).astype(jnp.float32) + pos_g
    kept = pos_g < CAP

    r_sub = lax.broadcasted_iota(jnp.int32, (BLK, T_PER), 0)

    def contrib(t):
        sel = (gcol_f == (r_sub + t * BLK).astype(jnp.float32)) & kept
        pt = sel.astype(jnp.bfloat16)
        return lax.dot_general(pt, comm_ref[t],
                               (((0,), (0,)), ((), ())),
                               preferred_element_type=jnp.float32)

    out_ref[...] = contrib(0)

    for t in range(1, H + 1):
        for c in range(N_CHUNK):
            r_desc[t - 1][c].wait_recv()
            if t < H:
                d = mk(t, t + 1, c, send_r.at[t, c], recv_r.at[t, c], right)
                d.start()
                r_desc[t][c] = d
        for c in range(N_CHUNK):
            if t <= H - 1:
                l_desc[t - 1][c].wait_recv()
            if t < H - 1:
                d = mk((N_DEV - t) % N_DEV, N_DEV - 1 - t, c,
                       send_l.at[t, c], recv_l.at[t, c], left)
                d.start()
                l_desc[t][c] = d
        out_ref[...] += contrib(t)
        if t < H:
            out_ref[...] += contrib(N_DEV - t)

    for row in r_desc + l_desc:
        for d in row:
            d.wait_send()


def kernel(x, router_W, route_idx, expert_W):
    del router_W
    my = lax.axis_index("i")

    my_pos = lax.rem(my, 4) * 4 + jnp.where(
        lax.rem(lax.rem(my, 4), 2) == 0, my // 4, 3 - my // 4)

    def dev_at(p):
        p = lax.rem(p + N_DEV, N_DEV)
        w = p // 4
        q = lax.rem(p, 4)
        z = jnp.where(lax.rem(w, 2) == 0, q, 3 - q)
        return z * 4 + w

    nbrs = jnp.stack([dev_at(my_pos + 1), dev_at(my_pos - 1),
                      my_pos]).astype(jnp.int32)

    route_row = route_idx.reshape(1, N_TOK)

    return pl.pallas_call(
        _body,
        out_shape=jax.ShapeDtypeStruct((T_PER, D_OUT), jnp.float32),
        in_specs=[
            pl.BlockSpec(memory_space=pltpu.MemorySpace.HBM),
            pl.BlockSpec(memory_space=pltpu.MemorySpace.HBM),
            pl.BlockSpec(memory_space=pltpu.VMEM),
            pl.BlockSpec(memory_space=pltpu.VMEM),
            pl.BlockSpec(memory_space=pltpu.SMEM),
        ],
        out_specs=pl.BlockSpec(memory_space=pltpu.VMEM),
        scratch_shapes=[
            pltpu.VMEM((N_TOK, D_IN), jnp.float32),
            pltpu.VMEM((E_PER, D_IN, D_OUT), jnp.float32),
            pltpu.VMEM((N_DEV, BLK, D_OUT), jnp.bfloat16),
            pltpu.SemaphoreType.DMA((1 + E_PER,)),
            pltpu.SemaphoreType.DMA((H, N_CHUNK)),
            pltpu.SemaphoreType.DMA((H, N_CHUNK)),
            pltpu.SemaphoreType.DMA((H - 1, N_CHUNK)),
            pltpu.SemaphoreType.DMA((H - 1, N_CHUNK)),
        ],
        compiler_params=pltpu.CompilerParams(collective_id=0),
    )(x, expert_W, route_row, route_idx, nbrs)


# device time: 40047 ns/iter; 1.0390x vs baseline; 1.0390x over previous
import jax
import jax.numpy as jnp
from jax import lax
from jax.experimental import pallas as pl
from jax.experimental.pallas import tpu as pltpu

N_DEV = 16
N_TOK = 2048
D_IN = 512
D_OUT = 1024
N_EXP = 64
E_PER = N_EXP // N_DEV
CAP = 25
CAP_PAD = 32
BLK = E_PER * CAP_PAD
T_PER = N_TOK // N_DEV
H = N_DEV // 2
N_CHUNK = 8
CH = BLK // N_CHUNK
CPE = CAP_PAD // CH

_PERM = (0, 4, 8, 12, 13, 9, 5, 1, 2, 6, 10, 14, 15, 11, 7, 3)
_RINGPOS = [0] * N_DEV
for _p, _d in enumerate(_PERM):
    _RINGPOS[_d] = _p


def _body(x_ref, ew_ref, route_ref, nbr_ref, out_ref,
          comm_ref, pos_ref, send_r, recv_r, send_l, recv_l):
    my = lax.axis_index("i")
    right = nbr_ref[0]
    left = nbr_ref[1]

    barrier_sem = pltpu.get_barrier_semaphore()
    for nbr in (left, right):
        pl.semaphore_signal(barrier_sem, inc=1, device_id=(nbr,),
                            device_id_type=pl.DeviceIdType.MESH)

    route_i = route_ref[...]
    oh = (lax.broadcasted_iota(jnp.int32, (N_EXP, N_TOK), 0)
          == route_i).astype(jnp.bfloat16)
    tri = (lax.broadcasted_iota(jnp.int32, (N_TOK, N_TOK), 0)
           < lax.broadcasted_iota(jnp.int32, (N_TOK, N_TOK), 1)
           ).astype(jnp.bfloat16)
    excl = jnp.dot(oh, tri, preferred_element_type=jnp.float32)
    pos_ref[...] = jnp.sum(excl * oh.astype(jnp.float32), axis=0,
                           keepdims=True)
    pos_f = pos_ref[...]

    row_id = lax.broadcasted_iota(jnp.int32, (BLK, N_TOK), 0)
    exp_of_r = my * E_PER + row_id // CAP_PAD
    c_of_r = row_id % CAP_PAD
    psrc = ((exp_of_r == route_i)
            & (c_of_r.astype(jnp.float32) == pos_f)
            & (c_of_r < CAP)).astype(jnp.float32)

    pl.semaphore_wait(barrier_sem, 2)

    def mk(src_slot, dst_slot, c, ssem, rsem, dev):
        return pltpu.make_async_remote_copy(
            src_ref=comm_ref.at[src_slot, pl.ds(c * CH, CH)],
            dst_ref=comm_ref.at[dst_slot, pl.ds(c * CH, CH)],
            send_sem=ssem,
            recv_sem=rsem,
            device_id=(dev,),
            device_id_type=pl.DeviceIdType.MESH,
        )

    r_desc = [[None] * N_CHUNK for _ in range(H)]
    l_desc = [[None] * N_CHUNK for _ in range(H - 1)]

    for s in range(E_PER):
        x_sel_s = jnp.dot(psrc[s * CAP_PAD:(s + 1) * CAP_PAD, :], x_ref[...],
                          preferred_element_type=jnp.float32)
        comm_ref[0, s * CAP_PAD:(s + 1) * CAP_PAD, :] = jnp.dot(
            x_sel_s, ew_ref[s],
            preferred_element_type=jnp.float32).astype(jnp.bfloat16)
        for c in range(s * CPE, (s + 1) * CPE):
            d = mk(0, 1, c, send_r.at[0, c], recv_r.at[0, c], right)
            d.start()
            r_desc[0][c] = d
            d = mk(0, N_DEV - 1, c, send_l.at[0, c], recv_l.at[0, c], left)
            d.start()
            l_desc[0][c] = d

    route_g = route_ref[:, pl.ds(my * T_PER, T_PER)]
    pos_g = pos_ref[:, pl.ds(my * T_PER, T_PER)]
    k_g = route_g // E_PER
    w_g = lax.rem(k_g, 4)
    z_g = k_g // 4
    rp_g = w_g * 4 + jnp.where(lax.rem(w_g, 2) == 0, z_g, 3 - z_g)
    my_pos = nbr_ref[2]
    slot_g = lax.rem(my_pos - rp_g + N_DEV, N_DEV)
    gcol_f = (slot_g * BLK
              + lax.rem(route_g, E_PER) * CAP_PAD).astype(jnp.float32) + pos_g
    kept = pos_g < CAP

    r_sub = lax.broadcasted_iota(jnp.int32, (BLK, T_PER), 0)

    def contrib(t):
        sel = (gcol_f == (r_sub + t * BLK).astype(jnp.float32)) & kept
        pt = sel.astype(jnp.bfloat16)
        return lax.dot_general(pt, comm_ref[t],
                               (((0,), (0,)), ((), ())),
                               preferred_element_type=jnp.float32)

    out_ref[...] = contrib(0)

    for t in range(1, H + 1):
        for c in range(N_CHUNK):
            r_desc[t - 1][c].wait_recv()
            if t < H:
                d = mk(t, t + 1, c, send_r.at[t, c], recv_r.at[t, c], right)
                d.start()
                r_desc[t][c] = d
        for c in range(N_CHUNK):
            if t <= H - 1:
                l_desc[t - 1][c].wait_recv()
            if t < H - 1:
                d = mk((N_DEV - t) % N_DEV, N_DEV - 1 - t, c,
                       send_l.at[t, c], recv_l.at[t, c], left)
                d.start()
                l_desc[t][c] = d
        out_ref[...] += contrib(t)
        if t < H:
            out_ref[...] += contrib(N_DEV - t)

    for row in r_desc + l_desc:
        for d in row:
            d.wait_send()


def kernel(x, router_W, route_idx, expert_W):
    del router_W
    my = lax.axis_index("i")

    my_pos = lax.rem(my, 4) * 4 + jnp.where(
        lax.rem(lax.rem(my, 4), 2) == 0, my // 4, 3 - my // 4)

    def dev_at(p):
        p = lax.rem(p + N_DEV, N_DEV)
        w = p // 4
        q = lax.rem(p, 4)
        z = jnp.where(lax.rem(w, 2) == 0, q, 3 - q)
        return z * 4 + w

    nbrs = jnp.stack([dev_at(my_pos + 1), dev_at(my_pos - 1),
                      my_pos]).astype(jnp.int32)

    route_row = route_idx.reshape(1, N_TOK)

    return pl.pallas_call(
        _body,
        out_shape=jax.ShapeDtypeStruct((T_PER, D_OUT), jnp.float32),
        in_specs=[
            pl.BlockSpec(memory_space=pltpu.VMEM),
            pl.BlockSpec(memory_space=pltpu.VMEM),
            pl.BlockSpec(memory_space=pltpu.VMEM),
            pl.BlockSpec(memory_space=pltpu.SMEM),
        ],
        out_specs=pl.BlockSpec(memory_space=pltpu.VMEM),
        scratch_shapes=[
            pltpu.VMEM((N_DEV, BLK, D_OUT), jnp.bfloat16),
            pltpu.VMEM((1, N_TOK), jnp.float32),
            pltpu.SemaphoreType.DMA((H, N_CHUNK)),
            pltpu.SemaphoreType.DMA((H, N_CHUNK)),
            pltpu.SemaphoreType.DMA((H - 1, N_CHUNK)),
            pltpu.SemaphoreType.DMA((H - 1, N_CHUNK)),
        ],
        compiler_params=pltpu.CompilerParams(collective_id=0),
    )(x, expert_W, route_row, nbrs)
